# baseline (speedup 1.0000x reference)
import functools

import jax
import jax.numpy as jnp
from jax import lax
from jax.experimental import pallas as pl
from jax.experimental.pallas import tpu as pltpu
from jax.experimental.pallas import tpu_sc as plsc

_S_TOPK = 64
_TAU = 0.1
_ETA = 0.5
_N_SOFT_LAYERS = 3
_L = 128
_BATCH = 32
_N = 1024


def _sd(x):
    return lax.rev(jnp.sort(x), (0,))


def _rev(x):
    return lax.rev(x, (0,))


def _merge2(a, b):
    br = _rev(b)
    return [_sd(jnp.maximum(a, br)), _sd(jnp.minimum(a, br))]


def _merge4(aa, bb):
    h = [jnp.maximum(aa[k], _rev(bb[1 - k])) for k in range(2)]
    lo = [jnp.minimum(aa[k], _rev(bb[1 - k])) for k in range(2)]

    def clean32(p):
        return [_sd(jnp.maximum(p[0], p[1])), _sd(jnp.minimum(p[0], p[1]))]

    return clean32(h) + clean32(lo)


def _top64(aa, bb):
    h = [jnp.maximum(aa[k], _rev(bb[3 - k])) for k in range(4)]
    u0, u2 = jnp.maximum(h[0], h[2]), jnp.minimum(h[0], h[2])
    u1, u3 = jnp.maximum(h[1], h[3]), jnp.minimum(h[1], h[3])
    w0, w1 = jnp.maximum(u0, u1), jnp.minimum(u0, u1)
    w2, w3 = jnp.maximum(u2, u3), jnp.minimum(u2, u3)
    return [_sd(w0), _sd(w1), _sd(w2), _sd(w3)]


def _sc_body(s_hbm, t_hbm, col_v, out_v):
    nc = 2
    wid = lax.axis_index("s") * nc + lax.axis_index("c")
    pltpu.sync_copy(s_hbm.at[wid], col_v)

    def run64(g):
        r = [_sd(col_v[pl.ds(64 * g + 16 * k, 16)]) for k in range(4)]
        return _merge4(_merge2(r[0], r[1]), _merge2(r[2], r[3]))

    acc = run64(0)
    for g in range(1, 16):
        acc = _top64(acc, run64(g))
    for k in range(4):
        out_v[pl.ds(16 * k, 16)] = acc[k]
    pltpu.sync_copy(out_v, t_hbm.at[wid])


def _make_sc_top64():
    mesh = plsc.VectorSubcoreMesh(core_axis_name="c", subcore_axis_name="s")
    return pl.kernel(
        _sc_body,
        out_type=jax.ShapeDtypeStruct((_BATCH, _S_TOPK), jnp.float32),
        mesh=mesh,
        scratch_types=[
            pltpu.VMEM((_N,), jnp.float32),
            pltpu.VMEM((_S_TOPK,), jnp.float32),
        ],
        compiler_params=pltpu.CompilerParams(needs_layout_passes=False),
    )


def _scan_sum(x, reverse):
    size = x.shape[-1]
    lane = lax.broadcasted_iota(jnp.int32, x.shape, x.ndim - 1)
    q = 1
    while q < size:
        if reverse:
            x = x + jnp.where(lane < size - q,
                              pltpu.roll(x, size - q, x.ndim - 1), 0.0)
        else:
            x = x + jnp.where(lane >= q, pltpu.roll(x, q, x.ndim - 1), 0.0)
        q *= 2
    return x


def _soft_mask(s0, t64):
    batch, n = s0.shape
    f32 = jnp.float32
    inv_tau = f32(1.0 / _TAU)
    t = jnp.concatenate([t64, jnp.zeros((batch, _L - _S_TOPK), f32)], axis=1)
    t0 = t[:, 0:1]
    t63 = t[:, _S_TOPK - 1:_S_TOPK]

    lane = lax.broadcasted_iota(jnp.int32, (batch, _L), 1)
    msk64 = lane < _S_TOPK
    craw = jnp.maximum(jnp.exp((t - t0) * inv_tau), f32(1e-30))
    c = jnp.where(msk64, craw, f32(0.0))
    cinv = jnp.where(msk64, f32(1.0) / craw, f32(0.0))
    c63 = c[:, _S_TOPK - 1:_S_TOPK]

    w_j = jnp.exp((s0 - t0) * inv_tau)
    below = s0 < t63
    vw = jnp.sum(jnp.where(below, w_j, f32(0.0)), axis=1, keepdims=True)
    n_ge = jnp.sum(jnp.where(below, f32(0.0), f32(1.0)), axis=1,
                   keepdims=True)

    s_ge = _scan_sum(c, reverse=True)
    s_lt = jnp.where(lane >= 1, pltpu.roll(_scan_sum(cinv, reverse=False),
                                           1, 1), f32(0.0))
    z = (vw + (n_ge - f32(_S_TOPK)) * c63 + s_ge) * cinv + c * s_lt
    zinv = jnp.where(msk64, f32(1.0) / z, f32(0.0))
    alpha = zinv * cinv
    beta = zinv * c
    a_tot = jnp.sum(alpha, axis=1, keepdims=True)
    b_sfx = jnp.where(lane < _L - 1,
                      pltpu.roll(_scan_sum(beta, reverse=True), _L - 1, 1),
                      f32(0.0))
    mval = c * _scan_sum(alpha, reverse=False) + cinv * b_sfx

    m1 = jnp.where(lane < _L - 1, pltpu.roll(mval, _L - 1, 1), f32(0.0))
    dmv = jnp.where(lane < _S_TOPK - 1, m1 - mval, f32(0.0))
    dmv3 = dmv[:, :_S_TOPK, None]
    tcmp = t[:, :_S_TOPK, None]
    cbig = s0[:, None, :] < tcmp
    acc = mval[:, 0:1] + jnp.sum(jnp.where(cbig, dmv3, f32(0.0)), axis=1)
    return jnp.where(below, w_j * a_tot, acc)


def _tc_pre_body(y_ref, a_ref, w_ref, bt_ref, s_ref):
    f32 = jnp.float32
    bt = f32(_ETA) * lax.dot_general(y_ref[...], a_ref[...],
                                     (((1,), (0,)), ((), ())),
                                     preferred_element_type=f32)
    bt_ref[...] = bt
    s_ref[...] = jnp.abs(bt * w_ref[...])


def _tc_gram_body(a_ref, m_ref):
    f32 = jnp.float32
    a = a_ref[...]
    gram = lax.dot_general(a, a, (((0,), (0,)), ((), ())),
                           preferred_element_type=f32)
    n = a.shape[1]
    ii = lax.broadcasted_iota(jnp.int32, (n, n), 0)
    jj = lax.broadcasted_iota(jnp.int32, (n, n), 1)
    m_ref[...] = jnp.where(ii == jj, f32(1.0), f32(0.0)) - f32(_ETA) * gram


def _tc_mid_body(ht_ref, s_ref, t_ref, m_ref, bt_ref, w_ref, h2_ref, s2_ref):
    f32 = jnp.float32
    ht = ht_ref[...]
    xt = _soft_mask(s_ref[...], t_ref[...]) * ht
    h2 = bt_ref[...] + lax.dot_general(xt, m_ref[...],
                                       (((1,), (0,)), ((), ())),
                                       preferred_element_type=f32)
    h2_ref[...] = h2
    s2_ref[...] = jnp.abs(h2 * w_ref[...])


def _tc_fin_body(ht_ref, s_ref, t_ref, o_ref):
    ht = ht_ref[...]
    o_ref[...] = _soft_mask(s_ref[...], t_ref[...]) * ht


def kernel(Y, A, W):
    batch, _ = Y.shape
    _, n = A.shape
    f32 = jnp.float32
    w2 = W.reshape(1, n)
    sds = jax.ShapeDtypeStruct

    bt, s1 = pl.pallas_call(
        _tc_pre_body,
        out_shape=(sds((batch, n), f32), sds((batch, n), f32)),
    )(Y, A, w2)
    mm = pl.pallas_call(
        _tc_gram_body, out_shape=sds((n, n), f32))(A)

    sc_top = _make_sc_top64()
    mid = pl.pallas_call(
        _tc_mid_body,
        out_shape=(sds((batch, n), f32), sds((batch, n), f32)),
    )
    fin = pl.pallas_call(_tc_fin_body, out_shape=sds((batch, n), f32))

    t1 = sc_top(s1)
    h2, s2 = mid(bt, s1, t1, mm, bt, w2)
    t2 = sc_top(s2)
    h3, s3 = mid(h2, s2, t2, mm, bt, w2)
    t3 = sc_top(s3)
    return fin(h3, s3, t3)
